# Initial kernel scaffold; baseline (speedup 1.0000x reference)
import functools

import jax
import jax.numpy as jnp
from jax import lax
from jax.experimental import pallas as pl
from jax.experimental.pallas import tpu as pltpu
from jax.experimental.pallas import tpu_sc as plsc

NUM_LEVELS = 8
LEVEL_DIM = 2
BASE_RES = 16
T = 2 ** 19
MASK = T - 1
P32 = jnp.int32(2654435761 - 2 ** 32)

BATCH = 1048576
NC, NS = 2, 16
NW = NC * NS
OUT_D = 3 * NUM_LEVELS * LEVEL_DIM


def _build(batch, chunk, interpret=False):
    ppw = batch // NW
    nch = ppw // chunk
    c = chunk
    sl = 128
    nsl = (4 * c) // sl
    mesh = plsc.VectorSubcoreMesh(core_axis_name="c", subcore_axis_name="s")

    def body(x_hbm, y_hbm, z_hbm, txy, tyz, tzx, out_hbm,
             xv, yv, zv, idxb, wbuf, rows, outb, sem):
        wid = lax.axis_index("s") * NC + lax.axis_index("c")
        lanes = lax.iota(jnp.int32, 16)

        @pl.loop(0, nch)
        def _chunk(t):
            base = wid * ppw + t * c
            pltpu.sync_copy(x_hbm.at[pl.ds(base, c)], xv)
            pltpu.sync_copy(y_hbm.at[pl.ds(base, c)], yv)
            pltpu.sync_copy(z_hbm.at[pl.ds(base, c)], zv)
            planes = ((xv, yv, txy), (yv, zv, tyz), (zv, xv, tzx))
            for plane, (ca, cb, tbl) in enumerate(planes):
                for l in range(NUM_LEVELS):
                    res = float(BASE_RES * (2 ** l))
                    lbase = l * T

                    @pl.loop(0, c // 16)
                    def _idx(j, ca=ca, cb=cb, res=res, lbase=lbase):
                        s = j * 16
                        a = ca[pl.ds(s, 16)] * res
                        b = cb[pl.ds(s, 16)] * res
                        ia = a.astype(jnp.int32)
                        ib = b.astype(jnp.int32)
                        fa = a - ia.astype(jnp.float32)
                        fb = b - ib.astype(jnp.float32)
                        hb0 = ib * P32
                        hb1 = hb0 + P32
                        ia1 = ia + 1
                        idxb[pl.ds(s, 16)] = ((ia ^ hb0) & MASK) + lbase
                        idxb[pl.ds(c + s, 16)] = ((ia ^ hb1) & MASK) + lbase
                        idxb[pl.ds(2 * c + s, 16)] = ((ia1 ^ hb0) & MASK) + lbase
                        idxb[pl.ds(3 * c + s, 16)] = ((ia1 ^ hb1) & MASK) + lbase
                        ga = 1.0 - fa
                        gb = 1.0 - fb
                        wbuf[pl.ds(s, 16)] = ga * gb
                        wbuf[pl.ds(c + s, 16)] = ga * fb
                        wbuf[pl.ds(2 * c + s, 16)] = fa * gb
                        wbuf[pl.ds(3 * c + s, 16)] = fa * fb

                    descs = []
                    for k in range(nsl):
                        descs.append(pltpu.async_copy(
                            tbl.at[idxb.at[pl.ds(k * sl, sl)]],
                            rows.at[pl.ds(k * sl, sl)], sem))
                    for d in descs:
                        d.wait()

                    colc = jnp.int32(plane * 2 * NUM_LEVELS + l * LEVEL_DIM)

                    @pl.loop(0, 2 * c // 16)
                    def _acc(j, colc=colc):
                        s2 = j * 16
                        pv = jnp.right_shift(s2 + lanes, 1)
                        di = lanes & 1
                        w0 = plsc.load_gather(wbuf, [pv])
                        w1 = plsc.load_gather(wbuf, [pv + c])
                        w2 = plsc.load_gather(wbuf, [pv + 2 * c])
                        w3 = plsc.load_gather(wbuf, [pv + 3 * c])
                        r0 = plsc.load_gather(rows, [pv, di])
                        r1 = plsc.load_gather(rows, [pv + c, di])
                        r2 = plsc.load_gather(rows, [pv + 2 * c, di])
                        r3 = plsc.load_gather(rows, [pv + 3 * c, di])
                        acc = r0 * w0 + r1 * w1 + r2 * w2 + r3 * w3
                        plsc.store_scatter(outb, [pv, colc + di], acc)

            pltpu.sync_copy(outb, out_hbm.at[pl.ds(base, c)])

    return pl.kernel(
        body,
        out_type=jax.ShapeDtypeStruct((batch, OUT_D), jnp.float32),
        mesh=mesh,
        scratch_types=[
            pltpu.VMEM((c,), jnp.float32),
            pltpu.VMEM((c,), jnp.float32),
            pltpu.VMEM((c,), jnp.float32),
            pltpu.VMEM((4 * c,), jnp.int32),
            pltpu.VMEM((4 * c,), jnp.float32),
            pltpu.VMEM((4 * c, LEVEL_DIM), jnp.float32),
            pltpu.VMEM((c, OUT_D), jnp.float32),
            pltpu.SemaphoreType.DMA,
        ],
        interpret=interpret,
    )


@functools.cache
def _get_kernel():
    return _build(BATCH, 512)


def kernel(inputs, table_xy, table_yz, table_zx):
    xs = inputs[:, 0]
    ys = inputs[:, 1]
    zs = inputs[:, 2]
    t0 = table_xy.reshape(NUM_LEVELS * T, LEVEL_DIM)
    t1 = table_yz.reshape(NUM_LEVELS * T, LEVEL_DIM)
    t2 = table_zx.reshape(NUM_LEVELS * T, LEVEL_DIM)
    return _get_kernel()(xs, ys, zs, t0, t1, t2)


# speedup vs baseline: 38.9684x; 38.9684x over previous
import functools

import jax
import jax.numpy as jnp
from jax import lax
from jax.experimental import pallas as pl
from jax.experimental.pallas import tpu as pltpu
from jax.experimental.pallas import tpu_sc as plsc

NUM_LEVELS = 8
LEVEL_DIM = 2
BASE_RES = 16
T = 2 ** 19
MASK = T - 1
P32 = jnp.int32(2654435761 - 2 ** 32)

BATCH = 1048576
NC, NS = 2, 16
NW = NC * NS
OUT_D = 3 * NUM_LEVELS * LEVEL_DIM


def _build(batch, chunk):
    ppw = batch // NW
    nch = ppw // chunk
    c = chunk
    sl = 2048
    nsl = (8 * c) // sl
    mesh = plsc.VectorSubcoreMesh(core_axis_name="c", subcore_axis_name="s",
                                  num_cores=NC, num_subcores=NS)

    def body(x_hbm, y_hbm, z_hbm, txy, tyz, tzx, out_hbm,
             xv, yv, zv, idxb, wbuf, rows, outb, sem):
        wid = lax.axis_index("s") * NC + lax.axis_index("c")
        lanes = lax.iota(jnp.int32, 16)

        @pl.loop(0, nch)
        def _chunk(t):
            base = wid * ppw + t * c
            pltpu.sync_copy(x_hbm.at[pl.ds(base, c)], xv)
            pltpu.sync_copy(y_hbm.at[pl.ds(base, c)], yv)
            pltpu.sync_copy(z_hbm.at[pl.ds(base, c)], zv)
            planes = ((xv, yv, txy), (yv, zv, tyz), (zv, xv, tzx))
            for plane, (ca, cb, tbl) in enumerate(planes):
                for l in range(NUM_LEVELS):
                    res = float(BASE_RES * (2 ** l))
                    lbase2 = 2 * l * T

                    @pl.loop(0, c // 16)
                    def _idx(j, ca=ca, cb=cb, res=res, lbase2=lbase2):
                        s = j * 16
                        a = ca[pl.ds(s, 16)] * res
                        b = cb[pl.ds(s, 16)] * res
                        ia = a.astype(jnp.int32)
                        ib = b.astype(jnp.int32)
                        fa = a - ia.astype(jnp.float32)
                        fb = b - ib.astype(jnp.float32)
                        hb0 = ib * P32
                        hb1 = hb0 + P32
                        ia1 = ia + 1
                        ipos = 2 * (s + lanes)
                        ipos1 = ipos + 1
                        for ci, (iac, hbc) in enumerate(
                                ((ia, hb0), (ia, hb1), (ia1, hb0), (ia1, hb1))):
                            e = 2 * ((iac ^ hbc) & MASK) + lbase2
                            off = ci * 2 * c
                            plsc.store_scatter(idxb, [off + ipos], e)
                            plsc.store_scatter(idxb, [off + ipos1], e + 1)
                        ga = 1.0 - fa
                        gb = 1.0 - fb
                        wbuf[pl.ds(s, 16)] = ga * gb
                        wbuf[pl.ds(c + s, 16)] = ga * fb
                        wbuf[pl.ds(2 * c + s, 16)] = fa * gb
                        wbuf[pl.ds(3 * c + s, 16)] = fa * fb

                    descs = []
                    for k in range(nsl):
                        descs.append(pltpu.async_copy(
                            tbl.at[idxb.at[pl.ds(k * sl, sl)]],
                            rows.at[pl.ds(k * sl, sl)], sem))
                    for d in descs:
                        d.wait()

                    colc = jnp.int32(plane * 2 * NUM_LEVELS + l * LEVEL_DIM)

                    @pl.loop(0, 2 * c // 16)
                    def _acc(j, colc=colc):
                        s2 = j * 16
                        pv = jnp.right_shift(s2 + lanes, 1)
                        di = lanes & 1
                        w0 = plsc.load_gather(wbuf, [pv])
                        w1 = plsc.load_gather(wbuf, [pv + c])
                        w2 = plsc.load_gather(wbuf, [pv + 2 * c])
                        w3 = plsc.load_gather(wbuf, [pv + 3 * c])
                        r0 = rows[pl.ds(s2, 16)]
                        r1 = rows[pl.ds(2 * c + s2, 16)]
                        r2 = rows[pl.ds(4 * c + s2, 16)]
                        r3 = rows[pl.ds(6 * c + s2, 16)]
                        acc = r0 * w0 + r1 * w1 + r2 * w2 + r3 * w3
                        plsc.store_scatter(outb, [pv, colc + di], acc)

            pltpu.sync_copy(outb, out_hbm.at[pl.ds(base, c)])

    return pl.kernel(
        body,
        out_type=jax.ShapeDtypeStruct((batch, OUT_D), jnp.float32),
        mesh=mesh,
        scratch_types=[
            pltpu.VMEM((c,), jnp.float32),
            pltpu.VMEM((c,), jnp.float32),
            pltpu.VMEM((c,), jnp.float32),
            pltpu.VMEM((8 * c,), jnp.int32),
            pltpu.VMEM((4 * c,), jnp.float32),
            pltpu.VMEM((8 * c,), jnp.float32),
            pltpu.VMEM((c, OUT_D), jnp.float32),
            pltpu.SemaphoreType.DMA,
        ],
        compiler_params=pltpu.CompilerParams(needs_layout_passes=False,
                                             use_tc_tiling_on_sc=False),
    )


@functools.cache
def _get_kernel():
    return _build(BATCH, 512)


def kernel(inputs, table_xy, table_yz, table_zx):
    xs = inputs[:, 0]
    ys = inputs[:, 1]
    zs = inputs[:, 2]
    t0 = table_xy.reshape(-1)
    t1 = table_yz.reshape(-1)
    t2 = table_zx.reshape(-1)
    return _get_kernel()(xs, ys, zs, t0, t1, t2)


# speedup vs baseline: 40.3941x; 1.0366x over previous
import functools

import jax
import jax.numpy as jnp
from jax import lax
from jax.experimental import pallas as pl
from jax.experimental.pallas import tpu as pltpu
from jax.experimental.pallas import tpu_sc as plsc

NUM_LEVELS = 8
LEVEL_DIM = 2
BASE_RES = 16
T = 2 ** 19
MASK = T - 1
P32 = jnp.int32(2654435761 - 2 ** 32)

BATCH = 1048576
NC, NS = 2, 16
NW = NC * NS
OUT_D = 3 * NUM_LEVELS * LEVEL_DIM


def _build(batch, chunk):
    ppw = batch // NW
    nch = ppw // chunk
    c = chunk
    sl = 2048
    nsl = (8 * c) // sl
    mesh = plsc.VectorSubcoreMesh(core_axis_name="c", subcore_axis_name="s",
                                  num_cores=NC, num_subcores=NS)

    def body(coords_hbm, txy, tyz, tzx, out_hbm,
             coords, idxb, wbuf, rows, outb, sem0, sem1):
        wid = lax.axis_index("s") * NC + lax.axis_index("c")
        lanes = lax.iota(jnp.int32, 16)
        sems = (sem0, sem1)

        @pl.loop(0, nch)
        def _chunk(t):
            base = wid * ppw + t * c
            pltpu.sync_copy(coords_hbm.at[pl.ds(base, c)], coords)

            steps = []
            for plane, (c0, c1, tbl) in enumerate(
                    ((0, 1, txy), (1, 2, tyz), (2, 0, tzx))):
                for l in range(NUM_LEVELS):
                    steps.append((plane, l, c0, c1, tbl))

            def fire(k):
                plane, l, c0, c1, tbl = steps[k]
                buf = k % 2
                res = float(BASE_RES * (2 ** l))
                lbase2 = 2 * l * T
                col0 = jnp.full((16,), c0, jnp.int32)
                col1 = jnp.full((16,), c1, jnp.int32)

                @pl.loop(0, c // 16)
                def _idx(j):
                    s = j * 16
                    rowv = s + lanes
                    a = plsc.load_gather(coords, [rowv, col0]) * res
                    b = plsc.load_gather(coords, [rowv, col1]) * res
                    ia = a.astype(jnp.int32)
                    ib = b.astype(jnp.int32)
                    fa = a - ia.astype(jnp.float32)
                    fb = b - ib.astype(jnp.float32)
                    hb0 = ib * P32
                    hb1 = hb0 + P32
                    ia1 = ia + 1
                    ipos = 2 * rowv
                    ipos1 = ipos + 1
                    for ci, (iac, hbc) in enumerate(
                            ((ia, hb0), (ia, hb1), (ia1, hb0), (ia1, hb1))):
                        e = 2 * ((iac ^ hbc) & MASK) + lbase2
                        off = buf * 8 * c + ci * 2 * c
                        plsc.store_scatter(idxb, [off + ipos], e)
                        plsc.store_scatter(idxb, [off + ipos1], e + 1)
                    ga = 1.0 - fa
                    gb = 1.0 - fb
                    woff = buf * 4 * c
                    wbuf[pl.ds(woff + s, 16)] = ga * gb
                    wbuf[pl.ds(woff + c + s, 16)] = ga * fb
                    wbuf[pl.ds(woff + 2 * c + s, 16)] = fa * gb
                    wbuf[pl.ds(woff + 3 * c + s, 16)] = fa * fb

                descs = []
                for k2 in range(nsl):
                    descs.append(pltpu.async_copy(
                        tbl.at[idxb.at[pl.ds(buf * 8 * c + k2 * sl, sl)]],
                        rows.at[pl.ds(buf * 8 * c + k2 * sl, sl)],
                        sems[buf]))
                return descs

            def accum(k, descs):
                plane, l, c0, c1, tbl = steps[k]
                buf = k % 2
                for d in descs:
                    d.wait()
                colc = jnp.int32(plane * 2 * NUM_LEVELS + l * LEVEL_DIM)
                roff = buf * 8 * c
                woff = buf * 4 * c

                @pl.loop(0, 2 * c // 16)
                def _acc(j):
                    s2 = j * 16
                    pv = jnp.right_shift(s2 + lanes, 1)
                    di = lanes & 1
                    w0 = plsc.load_gather(wbuf, [woff + pv])
                    w1 = plsc.load_gather(wbuf, [woff + c + pv])
                    w2 = plsc.load_gather(wbuf, [woff + 2 * c + pv])
                    w3 = plsc.load_gather(wbuf, [woff + 3 * c + pv])
                    r0 = rows[pl.ds(roff + s2, 16)]
                    r1 = rows[pl.ds(roff + 2 * c + s2, 16)]
                    r2 = rows[pl.ds(roff + 4 * c + s2, 16)]
                    r3 = rows[pl.ds(roff + 6 * c + s2, 16)]
                    acc = r0 * w0 + r1 * w1 + r2 * w2 + r3 * w3
                    plsc.store_scatter(outb, [pv, colc + di], acc)

            prev = fire(0)
            for k in range(1, len(steps)):
                cur = fire(k)
                accum(k - 1, prev)
                prev = cur
            accum(len(steps) - 1, prev)

            pltpu.sync_copy(outb, out_hbm.at[pl.ds(base, c)])

    return pl.kernel(
        body,
        out_type=jax.ShapeDtypeStruct((batch, OUT_D), jnp.float32),
        mesh=mesh,
        scratch_types=[
            pltpu.VMEM((c, 3), jnp.float32),
            pltpu.VMEM((2 * 8 * c,), jnp.int32),
            pltpu.VMEM((2 * 4 * c,), jnp.float32),
            pltpu.VMEM((2 * 8 * c,), jnp.float32),
            pltpu.VMEM((c, OUT_D), jnp.float32),
            pltpu.SemaphoreType.DMA,
            pltpu.SemaphoreType.DMA,
        ],
        compiler_params=pltpu.CompilerParams(needs_layout_passes=False,
                                             use_tc_tiling_on_sc=False),
    )


@functools.cache
def _get_kernel():
    return _build(BATCH, 512)


def kernel(inputs, table_xy, table_yz, table_zx):
    t0 = table_xy.reshape(-1)
    t1 = table_yz.reshape(-1)
    t2 = table_zx.reshape(-1)
    return _get_kernel()(inputs, t0, t1, t2)


# speedup vs baseline: 105.2722x; 2.6061x over previous
import functools

import jax
import jax.numpy as jnp
from jax import lax
from jax.experimental import pallas as pl
from jax.experimental.pallas import tpu as pltpu
from jax.experimental.pallas import tpu_sc as plsc

NUM_LEVELS = 8
LEVEL_DIM = 2
BASE_RES = 16
T = 2 ** 19
MASK = T - 1
P32 = jnp.int32(2654435761 - 2 ** 32)

BATCH = 1048576
NC, NS = 2, 16
NW = NC * NS
OUT_D = 3 * NUM_LEVELS * LEVEL_DIM


def _build(batch, chunk):
    ppw = batch // NW
    nch = ppw // chunk
    c = chunk
    sl = 2048
    nsl = (8 * c) // sl
    mesh = plsc.VectorSubcoreMesh(core_axis_name="c", subcore_axis_name="s",
                                  num_cores=NC, num_subcores=NS)

    def body(x_hbm, y_hbm, z_hbm, txy, tyz, tzx, out_hbm,
             xv, yv, zv, idxb, wbuf, rows, outb, sem0, sem1):
        wid = lax.axis_index("s") * NC + lax.axis_index("c")
        lanes = lax.iota(jnp.int32, 16)
        sems = (sem0, sem1)

        @pl.loop(0, nch)
        def _chunk(t):
            base = wid * ppw + t * c
            pltpu.sync_copy(x_hbm.at[pl.ds(base, c)], xv)
            pltpu.sync_copy(y_hbm.at[pl.ds(base, c)], yv)
            pltpu.sync_copy(z_hbm.at[pl.ds(base, c)], zv)

            steps = []
            for plane, (c0, c1, tbl) in enumerate(
                    ((xv, yv, txy), (yv, zv, tyz), (zv, xv, tzx))):
                for l in range(NUM_LEVELS):
                    steps.append((plane, l, c0, c1, tbl))

            def fire(k):
                plane, l, c0, c1, tbl = steps[k]
                buf = k % 2
                res = float(BASE_RES * (2 ** l))
                lbase2 = l * 2 * T

                @pl.loop(0, c // 16)
                def _idx(j):
                    s = j * 16
                    rowv = s + lanes
                    a = c0[pl.ds(s, 16)] * res
                    b = c1[pl.ds(s, 16)] * res
                    ia = a.astype(jnp.int32)
                    ib = b.astype(jnp.int32)
                    fa = a - ia.astype(jnp.float32)
                    fb = b - ib.astype(jnp.float32)
                    hb0 = ib * P32
                    hb1 = hb0 + P32
                    ia1 = ia + 1
                    ipos = 2 * rowv
                    ipos1 = ipos + 1
                    for ci, (iac, hbc) in enumerate(
                            ((ia, hb0), (ia, hb1), (ia1, hb0), (ia1, hb1))):
                        h = (iac ^ hbc) & MASK
                        e = h + (h & -128) + lbase2
                        off = buf * 8 * c + ci * 2 * c
                        plsc.store_scatter(idxb, [off + ipos], e)
                        plsc.store_scatter(idxb, [off + ipos1], e + 128)
                    ga = 1.0 - fa
                    gb = 1.0 - fb
                    woff = buf * 4 * c
                    wbuf[pl.ds(woff + s, 16)] = ga * gb
                    wbuf[pl.ds(woff + c + s, 16)] = ga * fb
                    wbuf[pl.ds(woff + 2 * c + s, 16)] = fa * gb
                    wbuf[pl.ds(woff + 3 * c + s, 16)] = fa * fb

                descs = []
                for k2 in range(nsl):
                    descs.append(pltpu.async_copy(
                        tbl.at[idxb.at[pl.ds(buf * 8 * c + k2 * sl, sl)]],
                        rows.at[pl.ds(buf * 8 * c + k2 * sl, sl)],
                        sems[buf]))
                return descs

            def accum(k, descs):
                plane, l, c0, c1, tbl = steps[k]
                buf = k % 2
                for d in descs:
                    d.wait()
                colc = jnp.int32(plane * 2 * NUM_LEVELS + l * LEVEL_DIM)
                roff = buf * 8 * c
                woff = buf * 4 * c

                @pl.loop(0, 2 * c // 16)
                def _acc(j):
                    s2 = j * 16
                    pv = jnp.right_shift(s2 + lanes, 1)
                    di = lanes & 1
                    w0 = plsc.load_gather(wbuf, [woff + pv])
                    w1 = plsc.load_gather(wbuf, [woff + c + pv])
                    w2 = plsc.load_gather(wbuf, [woff + 2 * c + pv])
                    w3 = plsc.load_gather(wbuf, [woff + 3 * c + pv])
                    r0 = rows[pl.ds(roff + s2, 16)]
                    r1 = rows[pl.ds(roff + 2 * c + s2, 16)]
                    r2 = rows[pl.ds(roff + 4 * c + s2, 16)]
                    r3 = rows[pl.ds(roff + 6 * c + s2, 16)]
                    acc = r0 * w0 + r1 * w1 + r2 * w2 + r3 * w3
                    plsc.store_scatter(outb, [pv, colc + di], acc)

            prev = fire(0)
            for k in range(1, len(steps)):
                cur = fire(k)
                accum(k - 1, prev)
                prev = cur
            accum(len(steps) - 1, prev)

            pltpu.sync_copy(outb, out_hbm.at[pl.ds(base, c)])

    return pl.kernel(
        body,
        out_type=jax.ShapeDtypeStruct((batch, OUT_D), jnp.float32),
        mesh=mesh,
        scratch_types=[
            pltpu.VMEM((c,), jnp.float32),
            pltpu.VMEM((c,), jnp.float32),
            pltpu.VMEM((c,), jnp.float32),
            pltpu.VMEM((2 * 8 * c,), jnp.int32),
            pltpu.VMEM((2 * 4 * c,), jnp.float32),
            pltpu.VMEM((2 * 8 * c,), jnp.float32),
            pltpu.VMEM((c, OUT_D), jnp.float32),
            pltpu.SemaphoreType.DMA,
            pltpu.SemaphoreType.DMA,
        ],
        compiler_params=pltpu.CompilerParams(needs_layout_passes=False,
                                             use_tc_tiling_on_sc=False),
    )


@functools.cache
def _get_kernel():
    return _build(BATCH, 512)


def _phys_flat(table):
    return table.reshape(NUM_LEVELS, T // 128, 128, LEVEL_DIM) \
                .transpose(0, 1, 3, 2).reshape(-1)


def kernel(inputs, table_xy, table_yz, table_zx):
    xs = inputs[:, 0]
    ys = inputs[:, 1]
    zs = inputs[:, 2]
    return _get_kernel()(xs, ys, zs, _phys_flat(table_xy),
                         _phys_flat(table_yz), _phys_flat(table_zx))


# speedup vs baseline: 181.1811x; 1.7211x over previous
import functools

import jax
import jax.numpy as jnp
from jax import lax
from jax.experimental import pallas as pl
from jax.experimental.pallas import tpu as pltpu
from jax.experimental.pallas import tpu_sc as plsc

NUM_LEVELS = 8
LEVEL_DIM = 2
BASE_RES = 16
T = 2 ** 19
MASK = T - 1
P32 = jnp.int32(2654435761 - 2 ** 32)
HI32 = jnp.int32(-65536)

BATCH = 1048576
NC, NS = 2, 16
NW = NC * NS
OUT_D = 3 * NUM_LEVELS * LEVEL_DIM


def _build(batch, chunk):
    ppw = batch // NW
    nch = ppw // chunk
    c = chunk
    sl = 2048
    nsl = max(1, (4 * c) // sl)
    mesh = plsc.VectorSubcoreMesh(core_axis_name="c", subcore_axis_name="s",
                                  num_cores=NC, num_subcores=NS)

    def body(x_hbm, y_hbm, z_hbm, txy, tyz, tzx, out_hbm,
             xv, yv, zv, idxb, wbuf, rows, outb, sem0, sem1):
        wid = lax.axis_index("s") * NC + lax.axis_index("c")
        lanes = lax.iota(jnp.int32, 16)
        sems = (sem0, sem1)

        @pl.loop(0, nch)
        def _chunk(t):
            base = wid * ppw + t * c
            pltpu.sync_copy(x_hbm.at[pl.ds(base, c)], xv)
            pltpu.sync_copy(y_hbm.at[pl.ds(base, c)], yv)
            pltpu.sync_copy(z_hbm.at[pl.ds(base, c)], zv)

            steps = []
            for plane, (c0, c1, tbl) in enumerate(
                    ((xv, yv, txy), (yv, zv, tyz), (zv, xv, tzx))):
                for l in range(NUM_LEVELS):
                    steps.append((plane, l, c0, c1, tbl))

            def fire(k):
                plane, l, c0, c1, tbl = steps[k]
                buf = k % 2
                res = float(BASE_RES * (2 ** l))
                lbase = l * T
                boff = buf * 4 * c

                @pl.loop(0, c // 16)
                def _idx(j):
                    s = j * 16
                    a = c0[pl.ds(s, 16)] * res
                    b = c1[pl.ds(s, 16)] * res
                    ia = a.astype(jnp.int32)
                    ib = b.astype(jnp.int32)
                    fa = a - ia.astype(jnp.float32)
                    fb = b - ib.astype(jnp.float32)
                    hb0 = ib * P32
                    hb1 = hb0 + P32
                    ia1 = ia + 1
                    idxb[pl.ds(boff + s, 16)] = ((ia ^ hb0) & MASK) + lbase
                    idxb[pl.ds(boff + c + s, 16)] = ((ia ^ hb1) & MASK) + lbase
                    idxb[pl.ds(boff + 2 * c + s, 16)] = ((ia1 ^ hb0) & MASK) + lbase
                    idxb[pl.ds(boff + 3 * c + s, 16)] = ((ia1 ^ hb1) & MASK) + lbase
                    ga = 1.0 - fa
                    gb = 1.0 - fb
                    wbuf[pl.ds(boff + s, 16)] = ga * gb
                    wbuf[pl.ds(boff + c + s, 16)] = ga * fb
                    wbuf[pl.ds(boff + 2 * c + s, 16)] = fa * gb
                    wbuf[pl.ds(boff + 3 * c + s, 16)] = fa * fb

                descs = []
                for k2 in range(nsl):
                    descs.append(pltpu.async_copy(
                        tbl.at[idxb.at[pl.ds(boff + k2 * sl, min(sl, 4 * c))]],
                        rows.at[pl.ds(boff + k2 * sl, min(sl, 4 * c))],
                        sems[buf]))
                return descs

            def accum(k, descs):
                plane, l, c0, c1, tbl = steps[k]
                buf = k % 2
                boff = buf * 4 * c
                for d in descs:
                    d.wait()
                col0 = jnp.full((16,), plane * 2 * NUM_LEVELS + l * LEVEL_DIM,
                                jnp.int32)
                col1 = col0 + 1

                @pl.loop(0, c // 16)
                def _acc(j):
                    s = j * 16
                    ptv = s + lanes
                    acc0 = jnp.zeros((16,), jnp.float32)
                    acc1 = jnp.zeros((16,), jnp.float32)
                    for ci in range(4):
                        r = rows[pl.ds(boff + ci * c + s, 16)]
                        f0 = plsc.bitcast(lax.shift_left(r, 16), jnp.float32)
                        f1 = plsc.bitcast(r & HI32, jnp.float32)
                        w = wbuf[pl.ds(boff + ci * c + s, 16)]
                        acc0 = acc0 + f0 * w
                        acc1 = acc1 + f1 * w
                    plsc.store_scatter(outb, [ptv, col0], acc0)
                    plsc.store_scatter(outb, [ptv, col1], acc1)

            prev = fire(0)
            for k in range(1, len(steps)):
                cur = fire(k)
                accum(k - 1, prev)
                prev = cur
            accum(len(steps) - 1, prev)

            pltpu.sync_copy(outb, out_hbm.at[pl.ds(base, c)])

    return pl.kernel(
        body,
        out_type=jax.ShapeDtypeStruct((batch, OUT_D), jnp.float32),
        mesh=mesh,
        scratch_types=[
            pltpu.VMEM((c,), jnp.float32),
            pltpu.VMEM((c,), jnp.float32),
            pltpu.VMEM((c,), jnp.float32),
            pltpu.VMEM((2 * 4 * c,), jnp.int32),
            pltpu.VMEM((2 * 4 * c,), jnp.float32),
            pltpu.VMEM((2 * 4 * c,), jnp.int32),
            pltpu.VMEM((c, OUT_D), jnp.float32),
            pltpu.SemaphoreType.DMA,
            pltpu.SemaphoreType.DMA,
        ],
        compiler_params=pltpu.CompilerParams(needs_layout_passes=False,
                                             use_tc_tiling_on_sc=False),
    )


@functools.cache
def _get_kernel():
    return _build(BATCH, 512)


def _pack(table):
    return lax.bitcast_convert_type(
        table.astype(jnp.bfloat16), jnp.int32).reshape(-1)


def kernel(inputs, table_xy, table_yz, table_zx):
    xs = inputs[:, 0]
    ys = inputs[:, 1]
    zs = inputs[:, 2]
    return _get_kernel()(xs, ys, zs, _pack(table_xy), _pack(table_yz),
                         _pack(table_zx))


# speedup vs baseline: 363.7186x; 2.0075x over previous
import functools

import jax
import jax.numpy as jnp
from jax import lax
from jax.experimental import pallas as pl
from jax.experimental.pallas import tpu as pltpu
from jax.experimental.pallas import tpu_sc as plsc

NUM_LEVELS = 8
LEVEL_DIM = 2
BASE_RES = 16
T = 2 ** 19
MASK = T - 1
P32 = jnp.int32(2654435761 - 2 ** 32)
HI32 = jnp.int32(-65536)

BATCH = 1048576
NC, NS = 2, 16
NW = NC * NS
OUT_D = 3 * NUM_LEVELS * LEVEL_DIM


def _pad128(n):
    return (n + 127) & ~127


TILE_LEVELS = (0, 1, 2, 3)
SP_LEVELS = (4,)
HBM_LEVELS = (5, 6, 7)

_tile_sizes = [_pad128((BASE_RES * 2 ** l + 1) ** 2) for l in TILE_LEVELS]
TILE_PLANE_SZ = sum(_tile_sizes)
TILE_OFF = {}
_off = 0
for _l, _s in zip(TILE_LEVELS, _tile_sizes):
    TILE_OFF[_l] = _off
    _off += _s

_sp_sizes = [_pad128((BASE_RES * 2 ** l + 1) ** 2) for l in SP_LEVELS]
SP_PLANE_SZ = sum(_sp_sizes)
SP_OFF = {}
_off = 0
for _l, _s in zip(SP_LEVELS, _sp_sizes):
    SP_OFF[_l] = _off
    _off += _s


def _build(batch, chunk):
    ppw = batch // NW
    nch = ppw // chunk
    c = chunk
    sl = 2048
    mesh = plsc.VectorSubcoreMesh(core_axis_name="c", subcore_axis_name="s",
                                  num_cores=NC, num_subcores=NS)

    def body(x_hbm, y_hbm, z_hbm, txy, tyz, tzx, out_hbm,
             xv, yv, zv, idxb, wbuf, rows, outb, gridt, grids, sem0, sem1):
        wid = lax.axis_index("s") * NC + lax.axis_index("c")
        sid = lax.axis_index("s")
        lanes = lax.iota(jnp.int32, 16)
        sems = (sem0, sem1)
        tables = (txy, tyz, tzx)

        def grid_fill(tbl, l, goff, n, dst_is_sp):
            res = BASE_RES * 2 ** l
            s_dim = res + 1
            lbase = l * T
            npad = _pad128(n)
            if dst_is_sp:
                nblk = npad // 128
                bpt = -(-nblk // NS)

                @pl.loop(0, bpt)
                def _b(i):
                    blk = sid * bpt + i

                    @pl.when(blk < nblk)
                    def _():
                        g0 = blk * 128

                        @pl.loop(0, 8)
                        def _i(j):
                            g = g0 + j * 16 + lanes
                            iy = g // s_dim
                            ix = g - iy * s_dim
                            e = ((ix ^ (iy * P32)) & MASK) + lbase
                            idxb[pl.ds(j * 16, 16)] = e

                        pltpu.async_copy(
                            tbl.at[idxb.at[pl.ds(0, 128)]],
                            rows.at[pl.ds(0, 128)], sem0).wait()
                        pltpu.sync_copy(
                            rows.at[pl.ds(0, 128)],
                            grids.at[pl.ds(goff + g0, 128)])
            else:
                off = 0
                while off < npad:
                    size = min(sl, npad - off)

                    @pl.loop(0, size // 16)
                    def _i(j, off=off):
                        g = off + j * 16 + lanes
                        iy = g // s_dim
                        ix = g - iy * s_dim
                        e = ((ix ^ (iy * P32)) & MASK) + lbase
                        idxb[pl.ds(j * 16, 16)] = e

                    pltpu.async_copy(
                        tbl.at[idxb.at[pl.ds(0, size)]],
                        gridt.at[pl.ds(goff + off, size)], sem0).wait()
                    off += size

        for plane, tbl in enumerate(tables):
            for l in TILE_LEVELS:
                n = (BASE_RES * 2 ** l + 1) ** 2
                grid_fill(tbl, l, plane * TILE_PLANE_SZ + TILE_OFF[l], n,
                          False)
            for l in SP_LEVELS:
                n = (BASE_RES * 2 ** l + 1) ** 2
                grid_fill(tbl, l, plane * SP_PLANE_SZ + SP_OFF[l], n, True)

        plsc.subcore_barrier()

        @pl.loop(0, nch)
        def _chunk(t):
            base = wid * ppw + t * c
            pltpu.sync_copy(x_hbm.at[pl.ds(base, c)], xv)
            pltpu.sync_copy(y_hbm.at[pl.ds(base, c)], yv)
            pltpu.sync_copy(z_hbm.at[pl.ds(base, c)], zv)

            dsteps, lsteps = [], []
            for plane, (c0, c1, tbl) in enumerate(
                    ((xv, yv, txy), (yv, zv, tyz), (zv, xv, tzx))):
                for l in range(NUM_LEVELS):
                    if l in TILE_LEVELS:
                        lsteps.append((plane, l, c0, c1))
                    else:
                        dsteps.append((plane, l, c0, c1, tbl))

            def unpack(r):
                f0 = plsc.bitcast(lax.shift_left(r, 16), jnp.float32)
                f1 = plsc.bitcast(r & HI32, jnp.float32)
                return f0, f1

            def corner_accum(j, rload, wload, plane, l):
                s = j * 16
                ptv = s + lanes
                acc0 = jnp.zeros((16,), jnp.float32)
                acc1 = jnp.zeros((16,), jnp.float32)
                for ci in range(4):
                    f0, f1 = unpack(rload(ci, s))
                    w = wload(ci, s)
                    acc0 = acc0 + f0 * w
                    acc1 = acc1 + f1 * w
                col0 = jnp.full((16,), plane * 2 * NUM_LEVELS + l * LEVEL_DIM,
                                jnp.int32)
                plsc.store_scatter(outb, [ptv, col0], acc0)
                plsc.store_scatter(outb, [ptv, col0 + 1], acc1)

            def fire(k):
                plane, l, c0, c1, tbl = dsteps[k]
                buf = k % 2
                res = float(BASE_RES * (2 ** l))
                boff = buf * 4 * c
                is_sp = l in SP_LEVELS
                if is_sp:
                    s_dim = BASE_RES * 2 ** l + 1
                    gbase = plane * SP_PLANE_SZ + SP_OFF[l]
                else:
                    lbase = l * T

                @pl.loop(0, c // 16)
                def _idx(j):
                    s = j * 16
                    a = c0[pl.ds(s, 16)] * res
                    b = c1[pl.ds(s, 16)] * res
                    ia = a.astype(jnp.int32)
                    ib = b.astype(jnp.int32)
                    fa = a - ia.astype(jnp.float32)
                    fb = b - ib.astype(jnp.float32)
                    if is_sp:
                        g = ib * s_dim + ia + gbase
                        idxb[pl.ds(boff + s, 16)] = g
                        idxb[pl.ds(boff + c + s, 16)] = g + s_dim
                        idxb[pl.ds(boff + 2 * c + s, 16)] = g + 1
                        idxb[pl.ds(boff + 3 * c + s, 16)] = g + s_dim + 1
                    else:
                        hb0 = ib * P32
                        hb1 = hb0 + P32
                        ia1 = ia + 1
                        idxb[pl.ds(boff + s, 16)] = ((ia ^ hb0) & MASK) + lbase
                        idxb[pl.ds(boff + c + s, 16)] = ((ia ^ hb1) & MASK) + lbase
                        idxb[pl.ds(boff + 2 * c + s, 16)] = ((ia1 ^ hb0) & MASK) + lbase
                        idxb[pl.ds(boff + 3 * c + s, 16)] = ((ia1 ^ hb1) & MASK) + lbase
                    ga = 1.0 - fa
                    gb = 1.0 - fb
                    wbuf[pl.ds(boff + s, 16)] = ga * gb
                    wbuf[pl.ds(boff + c + s, 16)] = ga * fb
                    wbuf[pl.ds(boff + 2 * c + s, 16)] = fa * gb
                    wbuf[pl.ds(boff + 3 * c + s, 16)] = fa * fb

                src = grids if is_sp else tbl
                descs = []
                for k2 in range((4 * c) // sl if 4 * c > sl else 1):
                    size = min(sl, 4 * c)
                    descs.append(pltpu.async_copy(
                        src.at[idxb.at[pl.ds(boff + k2 * size, size)]],
                        rows.at[pl.ds(boff + k2 * size, size)],
                        sems[buf]))
                return descs

            def accum(k, descs):
                plane, l, c0, c1, tbl = dsteps[k]
                buf = k % 2
                boff = buf * 4 * c
                for d in descs:
                    d.wait()

                def rload(ci, s):
                    return rows[pl.ds(boff + ci * c + s, 16)]

                def wload(ci, s):
                    return wbuf[pl.ds(boff + ci * c + s, 16)]

                @pl.loop(0, c // 16)
                def _acc(j):
                    corner_accum(j, rload, wload, plane, l)

            def local(k):
                plane, l, c0, c1 = lsteps[k]
                res = float(BASE_RES * (2 ** l))
                s_dim = BASE_RES * 2 ** l + 1
                gbase = plane * TILE_PLANE_SZ + TILE_OFF[l]

                @pl.loop(0, c // 16)
                def _loc(j):
                    s = j * 16
                    a = c0[pl.ds(s, 16)] * res
                    b = c1[pl.ds(s, 16)] * res
                    ia = a.astype(jnp.int32)
                    ib = b.astype(jnp.int32)
                    fa = a - ia.astype(jnp.float32)
                    fb = b - ib.astype(jnp.float32)
                    g = ib * s_dim + ia + gbase
                    ga = 1.0 - fa
                    gb = 1.0 - fb
                    ws = (ga * gb, ga * fb, fa * gb, fa * fb)
                    gs = (g, g + s_dim, g + 1, g + s_dim + 1)
                    acc0 = jnp.zeros((16,), jnp.float32)
                    acc1 = jnp.zeros((16,), jnp.float32)
                    for ci in range(4):
                        r = plsc.load_gather(gridt, [gs[ci]])
                        f0 = plsc.bitcast(lax.shift_left(r, 16), jnp.float32)
                        f1 = plsc.bitcast(r & HI32, jnp.float32)
                        w = ws[ci]
                        acc0 = acc0 + f0 * w
                        acc1 = acc1 + f1 * w
                    ptv = s + lanes
                    col0 = jnp.full((16,),
                                    plane * 2 * NUM_LEVELS + l * LEVEL_DIM,
                                    jnp.int32)
                    plsc.store_scatter(outb, [ptv, col0], acc0)
                    plsc.store_scatter(outb, [ptv, col0 + 1], acc1)

            prev = fire(0)
            for k in range(1, len(dsteps)):
                cur = fire(k)
                local(k - 1)
                accum(k - 1, prev)
                prev = cur
            local(len(dsteps) - 1)
            accum(len(dsteps) - 1, prev)

            pltpu.sync_copy(outb, out_hbm.at[pl.ds(base, c)])

    return pl.kernel(
        body,
        out_type=jax.ShapeDtypeStruct((batch, OUT_D), jnp.float32),
        mesh=mesh,
        scratch_types=[
            pltpu.VMEM((c,), jnp.float32),
            pltpu.VMEM((c,), jnp.float32),
            pltpu.VMEM((c,), jnp.float32),
            pltpu.VMEM((2 * 4 * c,), jnp.int32),
            pltpu.VMEM((2 * 4 * c,), jnp.float32),
            pltpu.VMEM((2 * 4 * c,), jnp.int32),
            pltpu.VMEM((c, OUT_D), jnp.float32),
            pltpu.VMEM((3 * TILE_PLANE_SZ,), jnp.int32),
            pltpu.VMEM_SHARED((3 * SP_PLANE_SZ,), jnp.int32),
            pltpu.SemaphoreType.DMA,
            pltpu.SemaphoreType.DMA,
        ],
        compiler_params=pltpu.CompilerParams(needs_layout_passes=False,
                                             use_tc_tiling_on_sc=False),
    )


@functools.cache
def _get_kernel():
    return _build(BATCH, 512)


def _pack(table):
    return lax.bitcast_convert_type(
        table.astype(jnp.bfloat16), jnp.int32).reshape(-1)


def kernel(inputs, table_xy, table_yz, table_zx):
    xs = inputs[:, 0]
    ys = inputs[:, 1]
    zs = inputs[:, 2]
    return _get_kernel()(xs, ys, zs, _pack(table_xy), _pack(table_yz),
                         _pack(table_zx))
